# baseline (speedup 1.0000x reference)
import math

import jax
import jax.numpy as jnp
from jax.experimental import pallas as pl
from jax.experimental.pallas import tpu as pltpu

_F = 26
_C = 1000
_D = 16
_K = _F * _C

_BM = 128


def _mm_body(z_ref, w_ref, o_ref):
    o_ref[:] = jnp.dot(z_ref[:], w_ref[:], preferred_element_type=jnp.float32)


def kernel(z, codebook):
    batch_shape = z.shape[:-1]
    m = math.prod(batch_shape)
    z_bf = z.reshape(m, _K).astype(jnp.bfloat16)
    w_bf = codebook.reshape(_K, _D).astype(jnp.bfloat16)

    out = pl.pallas_call(
        _mm_body,
        grid=(m // _BM,),
        in_specs=[
            pl.BlockSpec((_BM, _K), lambda i: (i, 0)),
            pl.BlockSpec((_K, _D), lambda i: (0, 0)),
        ],
        out_specs=pl.BlockSpec((_BM, _D), lambda i: (i, 0)),
        out_shape=jax.ShapeDtypeStruct((m, _D), jnp.float32),
        compiler_params=pltpu.CompilerParams(
            dimension_semantics=("parallel",),
            allow_input_fusion=(True, True),
        ),
    )(z_bf, w_bf)
    return out.reshape(*batch_shape, _D)


# speedup vs baseline: 1.0079x; 1.0079x over previous
import math

import jax
import jax.numpy as jnp
from jax.experimental import pallas as pl
from jax.experimental.pallas import tpu as pltpu

_F = 26
_C = 1000
_D = 16
_K = _F * _C

_BM = 128


def _mm_body(z_ref, w_ref, o_ref):
    o_ref[:] = jnp.dot(z_ref[:], w_ref[:], preferred_element_type=jnp.float32)


def kernel(z, codebook):
    batch_shape = z.shape[:-1]
    m = math.prod(batch_shape)
    z_bf = z.reshape(m, _K).astype(jnp.bfloat16)
    w_bf = codebook.reshape(_K, _D).astype(jnp.bfloat16)

    out = pl.pallas_call(
        _mm_body,
        grid=(m // _BM,),
        in_specs=[
            pl.BlockSpec((_BM, _K), lambda i: (i, 0)),
            pl.BlockSpec((_K, _D), lambda i: (0, 0)),
        ],
        out_specs=pl.BlockSpec((_BM, _D), lambda i: (i, 0)),
        out_shape=jax.ShapeDtypeStruct((m, _D), jnp.float32),
        compiler_params=pltpu.CompilerParams(
            dimension_semantics=("parallel",),
        ),
    )(z_bf, w_bf)
    return out.reshape(*batch_shape, _D)


# speedup vs baseline: 1.0909x; 1.0824x over previous
import math

import jax
import jax.numpy as jnp
from jax.experimental import pallas as pl
from jax.experimental.pallas import tpu as pltpu

_F = 26
_C = 1000
_D = 16
_K = _F * _C

_BM = 128


def _mm_body(z_ref, w_ref, o_ref):
    o_ref[:] = jnp.dot(
        z_ref[:],
        w_ref[:].astype(jnp.float32),
        preferred_element_type=jnp.float32,
    )


def kernel(z, codebook):
    batch_shape = z.shape[:-1]
    m = math.prod(batch_shape)
    z2 = z.reshape(m, _K)
    w = codebook.reshape(_K, _D).astype(jnp.bfloat16)

    out = pl.pallas_call(
        _mm_body,
        grid=(m // _BM,),
        in_specs=[
            pl.BlockSpec((_BM, _K), lambda i: (i, 0)),
            pl.BlockSpec((_K, _D), lambda i: (0, 0)),
        ],
        out_specs=pl.BlockSpec((_BM, _D), lambda i: (i, 0)),
        out_shape=jax.ShapeDtypeStruct((m, _D), jnp.float32),
        compiler_params=pltpu.CompilerParams(
            dimension_semantics=("parallel",),
        ),
    )(z2, w)
    return out.reshape(*batch_shape, _D)
